# baseline (speedup 1.0000x reference)
import functools

import jax
import jax.numpy as jnp
from jax import lax
from jax.experimental import pallas as pl
from jax.experimental.pallas import tpu as pltpu
from jax.experimental.pallas import tpu_sc as plsc

B, N, D, K = 4, 8192, 2048, 1024
NBLK = 4
BLK = N // NBLK
CK = 128
DN = (((1,), (0,)), ((), ()))
DNT = (((1,), (1,)), ((), ()))
HI = jax.lax.Precision.HIGHEST


def _scores_body(x_ref, w_ref, p_ref, s_scr):
    j = pl.program_id(1)
    x = x_ref[0]
    wT = w_ref[...]
    xb = x.astype(jnp.bfloat16)
    wb = wT.astype(jnp.bfloat16)
    acc = None
    for c in range(D // CK):
        part = lax.dot_general(wb[:, c * CK:(c + 1) * CK],
                               xb[:, c * CK:(c + 1) * CK],
                               DNT, preferred_element_type=jnp.float32)
        acc = part if acc is None else acc + part
    s_scr[:, pl.ds(j * BLK, BLK)] = acc

    @pl.when(j == NBLK - 1)
    def _():
        v = s_scr[...]
        m = jnp.max(v)
        u = jnp.exp(v - m)
        p_ref[0] = u / jnp.sum(u)


_scores_call = pl.pallas_call(
    _scores_body,
    grid=(B, NBLK),
    in_specs=[
        pl.BlockSpec((1, BLK, D), lambda b, j: (b, j, 0)),
        pl.BlockSpec((1, D), lambda b, j: (0, 0)),
    ],
    out_specs=pl.BlockSpec((1, 1, N), lambda b, j: (b, 0, 0)),
    out_shape=jax.ShapeDtypeStruct((B, 1, N), jnp.float32),
    scratch_shapes=[pltpu.VMEM((1, N), jnp.float32)],
)


NCH = N // K


def _topk_body(p_ref, idx_ref, idxg_ref, wc_ref):
    b = pl.program_id(0)
    p = p_ref[0]

    def bis(_, lohi):
        lo, hi = lohi
        mid = lo + (hi - lo) // 2
        midf = lax.bitcast_convert_type(jnp.zeros((1, N), jnp.int32) + mid,
                                        jnp.float32)
        cnt = jnp.sum((p >= midf).astype(jnp.float32))
        ge = cnt >= float(K)
        return (jnp.where(ge, mid, lo), jnp.where(ge, hi, mid))

    lo, hi = lax.fori_loop(0, 31, bis,
                           (jnp.int32(0), jnp.int32(0x7F800000)))
    tstar = lax.bitcast_convert_type(jnp.zeros((1, N), jnp.int32) + lo,
                                     jnp.float32)
    gt_m = (p > tstar).astype(jnp.float32)
    eq_m = (p == tstar).astype(jnp.float32)
    n_gt = jnp.sum(gt_m)
    n_tie = float(K) - n_gt

    iota_r = lax.broadcasted_iota(jnp.int32, (K, K), 0)
    iota_e = iota_r.astype(jnp.float32)
    upper = (iota_r < lax.broadcasted_iota(jnp.int32, (K, K), 1)
             ).astype(jnp.float32)

    def excl_prefix(mask):
        parts = []
        carry = jnp.float32(0.0)
        for c in range(NCH):
            mc = mask[:, c * K:(c + 1) * K]
            pref = lax.dot_general(mc, upper, DN,
                                   preferred_element_type=jnp.float32)
            parts.append(pref + carry)
            carry = carry + jnp.sum(mc)
        return jnp.concatenate(parts, axis=1)

    tie_pref = excl_prefix(eq_m)
    sel = gt_m + eq_m * (tie_pref < n_tie).astype(jnp.float32)
    pos = excl_prefix(sel)
    pos_m = jnp.where(sel > 0.5, pos, float(N))

    p_col = None
    e_col = None
    lane = lax.broadcasted_iota(jnp.int32, (1, K), 1).astype(jnp.float32)
    for c in range(NCH):
        pmc = pos_m[:, c * K:(c + 1) * K]
        Mc = (iota_e == pmc).astype(jnp.float32)
        pc = lax.dot_general(Mc, p[:, c * K:(c + 1) * K], DNT,
                             precision=HI, preferred_element_type=jnp.float32)
        ec = lax.dot_general(Mc, lane + float(c * K), DNT,
                             precision=HI, preferred_element_type=jnp.float32)
        p_col = pc if p_col is None else p_col + pc
        e_col = ec if e_col is None else e_col + ec

    p_row = jnp.swapaxes(p_col, 0, 1)
    e_row = jnp.swapaxes(e_col, 0, 1)
    gt = ((p_row > p_col) |
          ((p_row == p_col) & (e_row < e_col))).astype(jnp.float32)
    rank_col = jnp.sum(gt, axis=1, keepdims=True)
    rank_row = jnp.swapaxes(rank_col, 0, 1)
    M2 = (iota_e == rank_row).astype(jnp.float32)
    idx_sorted = lax.dot_general(M2, e_row, DNT, precision=HI,
                                 preferred_element_type=jnp.float32)
    idx_ref[0] = jnp.swapaxes(idx_sorted, 0, 1).astype(jnp.int32)

    u2 = jnp.exp(p_col - jnp.max(p))
    wc = u2 / jnp.sum(u2)
    wc_ref[0] = jnp.swapaxes(wc, 0, 1)
    idxg_ref[0] = (e_row + float(N) * b.astype(jnp.float32)).astype(jnp.int32)


_topk_call = pl.pallas_call(
    _topk_body,
    grid=(B,),
    in_specs=[pl.BlockSpec((1, 1, N), lambda b: (b, 0, 0))],
    out_specs=[
        pl.BlockSpec((1, 1, K), lambda b: (b, 0, 0)),
        pl.BlockSpec((1, 1, K), lambda b: (b, 0, 0)),
        pl.BlockSpec((1, 1, K), lambda b: (b, 0, 0)),
    ],
    out_shape=[
        jax.ShapeDtypeStruct((B, 1, K), jnp.int32),
        jax.ShapeDtypeStruct((B, 1, K), jnp.int32),
        jax.ShapeDtypeStruct((B, 1, K), jnp.float32),
    ],
)


NW = 32
RPW = (B * K) // NW
CH = 32


@functools.partial(
    pl.kernel,
    mesh=plsc.VectorSubcoreMesh(core_axis_name="c", subcore_axis_name="s"),
    out_type=jax.ShapeDtypeStruct((B * K, D), jnp.float32),
    scratch_types=[
        pltpu.VMEM((CH,), jnp.int32),
        pltpu.VMEM((CH, D), jnp.float32),
        pltpu.SemaphoreType.DMA,
    ],
)
def _gather_call(x_hbm, idx_hbm, out_hbm, idx_v, rows_v, sem):
    wid = lax.axis_index("s") * 2 + lax.axis_index("c")
    base = wid * RPW
    for c in range(RPW // CH):
        start = base + c * CH
        pltpu.sync_copy(idx_hbm.at[pl.ds(start, CH)], idx_v)
        pltpu.async_copy(x_hbm.at[idx_v], rows_v, sem).wait()
        pltpu.sync_copy(rows_v, out_hbm.at[pl.ds(start, CH)])


def _wsum_body(g_ref, wc_ref, o_ref, emb_scr):
    b = pl.program_id(0)
    part = lax.dot_general(wc_ref[0], g_ref[0], DN, precision=HI,
                           preferred_element_type=jnp.float32)
    emb_scr[pl.ds(b, 1), :] = part

    @pl.when(b == B - 1)
    def _():
        e = emb_scr[...]
        sq = jnp.sum(e * e)
        o_ref[...] = e * lax.rsqrt(jnp.maximum(sq, 1e-12))


_wsum_call = pl.pallas_call(
    _wsum_body,
    grid=(B,),
    in_specs=[
        pl.BlockSpec((1, K, D), lambda b: (b, 0, 0)),
        pl.BlockSpec((1, 1, K), lambda b: (b, 0, 0)),
    ],
    out_specs=pl.BlockSpec((B, D), lambda b: (0, 0)),
    out_shape=jax.ShapeDtypeStruct((B, D), jnp.float32),
    scratch_shapes=[pltpu.VMEM((B, D), jnp.float32)],
)


def kernel(input_data, kernel):
    wT = kernel.reshape(1, D)
    p3 = _scores_call(input_data, wT)
    idx_s, idxg, wc = _topk_call(p3)
    x2 = input_data.reshape(B * N, D)
    gathered = _gather_call(x2, idxg.reshape(B * K))
    emb = _wsum_call(gathered.reshape(B, K, D), wc)
    return (idx_s.reshape(B, K),
            p3.reshape(B, N, 1),
            emb)


# speedup vs baseline: 1.0070x; 1.0070x over previous
import functools

import jax
import jax.numpy as jnp
from jax import lax
from jax.experimental import pallas as pl
from jax.experimental.pallas import tpu as pltpu
from jax.experimental.pallas import tpu_sc as plsc

B, N, D, K = 4, 8192, 2048, 1024
NBLK = 4
BLK = N // NBLK
CK = 128
DN = (((1,), (0,)), ((), ()))
DNT = (((1,), (1,)), ((), ()))
HI = jax.lax.Precision.HIGHEST


def _scores_body(x_ref, w_ref, p_ref, s_scr):
    j = pl.program_id(1)
    x = x_ref[0]
    wT = w_ref[...]
    xb = x.astype(jnp.bfloat16)
    wb = wT.astype(jnp.bfloat16)
    acc = None
    for c in range(D // CK):
        part = lax.dot_general(wb[:, c * CK:(c + 1) * CK],
                               xb[:, c * CK:(c + 1) * CK],
                               DNT, preferred_element_type=jnp.float32)
        acc = part if acc is None else acc + part
    s_scr[:, pl.ds(j * BLK, BLK)] = acc

    @pl.when(j == NBLK - 1)
    def _():
        v = s_scr[...]
        m = jnp.max(v)
        u = jnp.exp(v - m)
        p_ref[0] = u / jnp.sum(u)


_scores_call = pl.pallas_call(
    _scores_body,
    grid=(B, NBLK),
    in_specs=[
        pl.BlockSpec((1, BLK, D), lambda b, j: (b, j, 0)),
        pl.BlockSpec((1, D), lambda b, j: (0, 0)),
    ],
    out_specs=pl.BlockSpec((1, 1, N), lambda b, j: (b, 0, 0)),
    out_shape=jax.ShapeDtypeStruct((B, 1, N), jnp.float32),
    scratch_shapes=[pltpu.VMEM((1, N), jnp.float32)],
)


NCH = N // K


def _topk_body(p_ref, idx_ref, idxg_ref, wc_ref):
    b = pl.program_id(0)
    p = p_ref[0]

    def bis(_, lohi):
        lo, hi = lohi
        mid = lo + (hi - lo) // 2
        midf = lax.bitcast_convert_type(jnp.zeros((1, N), jnp.int32) + mid,
                                        jnp.float32)
        cnt = jnp.sum((p >= midf).astype(jnp.float32))
        ge = cnt >= float(K)
        return (jnp.where(ge, mid, lo), jnp.where(ge, hi, mid))

    lo, hi = lax.fori_loop(0, 31, bis,
                           (jnp.int32(0), jnp.int32(0x7F800000)))
    tstar = lax.bitcast_convert_type(jnp.zeros((1, N), jnp.int32) + lo,
                                     jnp.float32)
    gt_m = (p > tstar).astype(jnp.float32)
    eq_m = (p == tstar).astype(jnp.float32)
    n_gt = jnp.sum(gt_m)
    n_tie = float(K) - n_gt

    iota_r = lax.broadcasted_iota(jnp.int32, (K, K), 0)
    iota_e = iota_r.astype(jnp.float32)
    upper = (iota_r < lax.broadcasted_iota(jnp.int32, (K, K), 1)
             ).astype(jnp.float32)

    def excl_prefix(mask):
        parts = []
        carry = jnp.float32(0.0)
        for c in range(NCH):
            mc = mask[:, c * K:(c + 1) * K]
            pref = lax.dot_general(mc, upper, DN,
                                   preferred_element_type=jnp.float32)
            parts.append(pref + carry)
            carry = carry + jnp.sum(mc)
        return jnp.concatenate(parts, axis=1)

    tie_pref = excl_prefix(eq_m)
    sel = gt_m + eq_m * (tie_pref < n_tie).astype(jnp.float32)
    pos = excl_prefix(sel)
    pos_m = jnp.where(sel > 0.5, pos, float(N))

    p_col = None
    e_col = None
    lane = lax.broadcasted_iota(jnp.int32, (1, K), 1).astype(jnp.float32)
    for c in range(NCH):
        pmc = pos_m[:, c * K:(c + 1) * K]
        Mc = (iota_e == pmc).astype(jnp.float32)
        pc = lax.dot_general(Mc, p[:, c * K:(c + 1) * K], DNT,
                             precision=HI, preferred_element_type=jnp.float32)
        ec = lax.dot_general(Mc, lane + float(c * K), DNT,
                             precision=HI, preferred_element_type=jnp.float32)
        p_col = pc if p_col is None else p_col + pc
        e_col = ec if e_col is None else e_col + ec

    p_row = jnp.swapaxes(p_col, 0, 1)
    e_row = jnp.swapaxes(e_col, 0, 1)
    gt = ((p_row > p_col) |
          ((p_row == p_col) & (e_row < e_col))).astype(jnp.float32)
    rank_col = jnp.sum(gt, axis=1, keepdims=True)
    rank_row = jnp.swapaxes(rank_col, 0, 1)
    M2 = (iota_e == rank_row).astype(jnp.float32)
    idx_sorted = lax.dot_general(M2, e_row, DNT, precision=HI,
                                 preferred_element_type=jnp.float32)
    idx_ref[0] = jnp.swapaxes(idx_sorted, 0, 1).astype(jnp.int32)

    u2 = jnp.exp(p_col - jnp.max(p))
    wc = u2 / jnp.sum(u2)
    wc_ref[0] = jnp.broadcast_to(wc, (K, 16))
    idxg_ref[0] = (e_row + float(N) * b.astype(jnp.float32)).astype(jnp.int32)


_topk_call = pl.pallas_call(
    _topk_body,
    grid=(B,),
    in_specs=[pl.BlockSpec((1, 1, N), lambda b: (b, 0, 0))],
    out_specs=[
        pl.BlockSpec((1, 1, K), lambda b: (b, 0, 0)),
        pl.BlockSpec((1, 1, K), lambda b: (b, 0, 0)),
        pl.BlockSpec((1, K, 16), lambda b: (b, 0, 0)),
    ],
    out_shape=[
        jax.ShapeDtypeStruct((B, 1, K), jnp.int32),
        jax.ShapeDtypeStruct((B, 1, K), jnp.int32),
        jax.ShapeDtypeStruct((B, K, 16), jnp.float32),
    ],
)


NW = 32
RPW = (B * K) // NW
CH = 32
VL = 16


@functools.partial(
    pl.kernel,
    mesh=plsc.VectorSubcoreMesh(core_axis_name="c", subcore_axis_name="s"),
    out_type=jax.ShapeDtypeStruct((NW, D), jnp.float32),
    scratch_types=[
        pltpu.VMEM((CH,), jnp.int32),
        pltpu.VMEM((CH, D), jnp.float32),
        pltpu.VMEM((1, D), jnp.float32),
        pltpu.VMEM((CH, 16), jnp.float32),
        pltpu.SemaphoreType.DMA,
    ],
)
def _gather_call(x_hbm, idx_hbm, w_hbm, out_hbm, idx_v, rows_v, acc_v, w_v,
                 sem):
    wid = lax.axis_index("s") * 2 + lax.axis_index("c")
    base = wid * RPW

    def zero(k, _):
        acc_v[0, pl.ds(k * VL, VL)] = jnp.zeros((VL,), jnp.float32)
        return 0

    lax.fori_loop(0, D // VL, zero, 0)
    for c in range(RPW // CH):
        start = base + c * CH
        pltpu.sync_copy(idx_hbm.at[pl.ds(start, CH)], idx_v)
        pltpu.sync_copy(w_hbm.at[pl.ds(start, CH)], w_v)
        pltpu.async_copy(x_hbm.at[idx_v], rows_v, sem).wait()

        def accum(k, _):
            o = k * VL
            a = acc_v[0, pl.ds(o, VL)]
            for r in range(CH):
                a = a + w_v[r] * rows_v[r, pl.ds(o, VL)]
            acc_v[0, pl.ds(o, VL)] = a
            return 0

        lax.fori_loop(0, D // VL, accum, 0)
    pltpu.sync_copy(acc_v, out_hbm.at[pl.ds(wid, 1)])


def _norm_body(p_ref, o_ref):
    part = p_ref[...]
    rows = [jnp.sum(part[8 * b:8 * (b + 1)], axis=0, keepdims=True)
            for b in range(B)]
    e = jnp.concatenate(rows, axis=0)
    sq = jnp.sum(e * e)
    o_ref[...] = e * lax.rsqrt(jnp.maximum(sq, 1e-12))


_norm_call = pl.pallas_call(
    _norm_body,
    in_specs=[pl.BlockSpec((NW, D), lambda: (0, 0))],
    out_specs=pl.BlockSpec((B, D), lambda: (0, 0)),
    out_shape=jax.ShapeDtypeStruct((B, D), jnp.float32),
)


def kernel(input_data, kernel):
    wT = kernel.reshape(1, D)
    p3 = _scores_call(input_data, wT)
    idx_s, idxg, wc = _topk_call(p3)
    x2 = input_data.reshape(B * N, D)
    partials = _gather_call(x2, idxg.reshape(B * K), wc.reshape(B * K, 16))
    emb = _norm_call(partials)
    return (idx_s.reshape(B, K),
            p3.reshape(B, N, 1),
            emb)
